# baseline (speedup 1.0000x reference)
import functools
import jax
import jax.numpy as jnp
import numpy as np
from jax import lax
from jax.experimental import pallas as pl
from jax.experimental.pallas import tpu as pltpu, tpu_sc as plsc

_SCALE = float(2.0 ** -2.5)

_B, _T = 128, 32768
_F = _T // 32
_TC_ROWS = 96
_SC_ROWS = _B - _TC_ROWS
_mesh = plsc.VectorSubcoreMesh(core_axis_name="c", subcore_axis_name="s")

_REL = np.concatenate(
    [
        np.array([33 * l + j - 32 * ((l + j) >= 32) for l in range(16)], np.int32)
        for j in range(32)
    ]
)


def _tc_body(x_ref, o_ref):
    xb = x_ref[...]
    xt = jnp.transpose(xb)
    s = xt.reshape(xt.shape[0] // 32, 32, xt.shape[1]).sum(axis=1) * _SCALE
    o_ref[...] = jnp.transpose(s)


def _tc_part(x):
    CB = 16384
    return pl.pallas_call(
        _tc_body,
        grid=(_T // CB,),
        in_specs=[pl.BlockSpec((_TC_ROWS, CB), lambda i: (0, i))],
        out_specs=pl.BlockSpec((_TC_ROWS, CB // 32), lambda i: (0, i)),
        out_shape=jax.ShapeDtypeStruct((_B, _F), jnp.float32),
        compiler_params=pltpu.CompilerParams(
            dimension_semantics=("parallel",),
        ),
    )(x)


@functools.partial(
    pl.kernel,
    out_type=jax.ShapeDtypeStruct((_SC_ROWS, _F), jnp.float32),
    mesh=_mesh,
    scratch_types=[
        pltpu.VMEM((_T,), jnp.float32),
        pltpu.VMEM((_F,), jnp.float32),
        pltpu.VMEM((32 * 16,), jnp.int32),
        pltpu.SemaphoreType.DMA,
    ],
    compiler_params=pltpu.CompilerParams(
        needs_layout_passes=False,
        skip_device_barrier=True,
    ),
)
def _sc_reduce(x_hbm, rel_hbm, o_hbm, buf, outb, relv, sem):
    wid = lax.axis_index("s") * 2 + lax.axis_index("c")
    row = _TC_ROWS + wid
    pltpu.sync_copy(rel_hbm, relv)
    rel = [relv[pl.ds(j * 16, 16)] for j in range(32)]
    pltpu.async_copy(x_hbm.at[row], buf, sem).wait()

    def g_body(g, _):
        base = g * 512
        a0 = plsc.load_gather(buf, [base + rel[0]])
        a1 = plsc.load_gather(buf, [base + rel[1]])
        a2 = plsc.load_gather(buf, [base + rel[2]])
        a3 = plsc.load_gather(buf, [base + rel[3]])
        for j in range(4, 32, 4):
            a0 = a0 + plsc.load_gather(buf, [base + rel[j]])
            a1 = a1 + plsc.load_gather(buf, [base + rel[j + 1]])
            a2 = a2 + plsc.load_gather(buf, [base + rel[j + 2]])
            a3 = a3 + plsc.load_gather(buf, [base + rel[j + 3]])
        outb[pl.ds(g * 16, 16)] = ((a0 + a1) + (a2 + a3)) * _SCALE
        return 0

    lax.fori_loop(0, _F // 16, g_body, 0)
    pltpu.sync_copy(outb, o_hbm.at[wid])


def kernel(x):
    full = _tc_part(x)
    bot = _sc_reduce(x, jnp.asarray(_REL))
    out = lax.dynamic_update_slice(full, bot, (_TC_ROWS, 0))
    return out[:, :, None]


# speedup vs baseline: 2.6510x; 2.6510x over previous
import jax
import jax.numpy as jnp
import numpy as np
from jax.experimental import pallas as pl
from jax.experimental.pallas import tpu as pltpu

_SCALE = float(2.0 ** -2.5)


def _body(x_ref, o_ref):
    xb = x_ref[...]
    xt = jnp.transpose(xb)
    s = xt.reshape(xt.shape[0] // 32, 32, xt.shape[1]).sum(axis=1) * _SCALE
    o_ref[...] = jnp.transpose(s)


def kernel(x):
    B, T = x.shape
    F = T // 32
    CB = 16384
    out = pl.pallas_call(
        _body,
        grid=(T // CB,),
        in_specs=[pl.BlockSpec((B, CB), lambda i: (0, i))],
        out_specs=pl.BlockSpec((B, CB // 32), lambda i: (0, i)),
        out_shape=jax.ShapeDtypeStruct((B, F), jnp.float32),
        compiler_params=pltpu.CompilerParams(
            dimension_semantics=("parallel",),
        ),
    )(x)
    return out[:, :, None]
